# baseline (speedup 1.0000x reference)
import functools

import jax
import jax.numpy as jnp
from jax import lax
from jax.experimental import pallas as pl
from jax.experimental.pallas import tpu as pltpu
from jax.experimental.pallas import tpu_sc as plsc

_G, _T, _E = 4, 8192, 64
_NTOK = _G * _T
_Z_COEF = 0.001
_AUX_COEF = 0.01


def _sc_expert_counts(idx0, idx1):
    info = plsc.get_sparse_core_info()
    nc, ns, lanes = info.num_cores, info.num_subcores, info.num_lanes
    nw = nc * ns
    per_w = _NTOK // nw
    mesh = plsc.VectorSubcoreMesh(core_axis_name="c", subcore_axis_name="s")

    @functools.partial(
        pl.kernel,
        mesh=mesh,
        out_type=jax.ShapeDtypeStruct((nw, _E), jnp.float32),
        compiler_params=pltpu.CompilerParams(needs_layout_passes=False),
        scratch_types=[
            pltpu.VMEM((per_w,), jnp.int32),
            pltpu.VMEM((per_w,), jnp.int32),
            pltpu.VMEM((lanes * _E,), jnp.float32),
            pltpu.VMEM((_E,), jnp.float32),
        ],
    )
    def hist_kernel(idx0_hbm, idx1_hbm, out_hbm, i0_v, i1_v, h_lane, h_row):
        wid = lax.axis_index("s") * nc + lax.axis_index("c")
        base = wid * per_w
        pltpu.sync_copy(idx0_hbm.at[pl.ds(base, per_w)], i0_v)
        pltpu.sync_copy(idx1_hbm.at[pl.ds(base, per_w)], i1_v)

        zeros = jnp.zeros((lanes,), jnp.float32)
        for r in range(lanes * _E // lanes):
            h_lane[pl.ds(r * lanes, lanes)] = zeros

        lane_base = lax.iota(jnp.int32, lanes) * _E
        ones = jnp.ones((lanes,), jnp.float32)

        def body(i, carry):
            v0 = i0_v[pl.ds(i * lanes, lanes)]
            v1 = i1_v[pl.ds(i * lanes, lanes)]
            plsc.addupdate_scatter(h_lane, [lane_base + v0], ones)
            plsc.addupdate_scatter(h_lane, [lane_base + v1], ones, mask=v1 != v0)
            return carry

        lax.fori_loop(0, per_w // lanes, body, 0)

        for c in range(_E // lanes):
            acc = h_lane[pl.ds(c * lanes, lanes)]
            for r in range(1, lanes):
                acc = acc + h_lane[pl.ds(r * _E + c * lanes, lanes)]
            h_row[pl.ds(c * lanes, lanes)] = acc

        pltpu.sync_copy(h_row, out_hbm.at[wid])

    return hist_kernel(idx0, idx1)


_TB = 512


def _tc_loss(counts, logits):
    ntb = _T // _TB
    rows_per_group = counts.shape[0] // _G

    def body(counts_ref, x_ref, out_ref, acc_ref):
        g = pl.program_id(0)
        t = pl.program_id(1)

        @pl.when((g == 0) & (t == 0))
        def _init():
            acc_ref[0] = 0.0
            acc_ref[1] = 0.0

        x = x_ref[0]
        m = jnp.max(x, axis=-1, keepdims=True)
        ex = jnp.exp(x - m)
        s = jnp.sum(ex, axis=-1, keepdims=True)
        log_z = m + jnp.log(s)
        z_part = jnp.sum(log_z * log_z)
        col_sum = jnp.sum(ex / s, axis=0)
        cnt = jnp.sum(counts_ref[...], axis=0)
        aux_part = jnp.sum(col_sum * cnt)
        acc_ref[0] += z_part
        acc_ref[1] += aux_part

        @pl.when((g == _G - 1) & (t == ntb - 1))
        def _final():
            z_loss = acc_ref[0] / (_G * _T)
            aux_loss = acc_ref[1] * (_E / (_G * _T * _T))
            total = _Z_COEF * z_loss + _AUX_COEF * aux_loss
            out_ref[...] = jnp.broadcast_to(total, (1, 1))

    return pl.pallas_call(
        body,
        grid=(_G, ntb),
        in_specs=[
            pl.BlockSpec((rows_per_group, _E), lambda g, t: (g, 0)),
            pl.BlockSpec((1, _TB, _E), lambda g, t: (g, t, 0)),
        ],
        out_specs=pl.BlockSpec((1, 1), lambda g, t: (0, 0)),
        out_shape=jax.ShapeDtypeStruct((1, 1), jnp.float32),
        scratch_shapes=[pltpu.SMEM((2,), jnp.float32)],
    )(counts, logits)


def kernel(router_logits, expert_indexes):
    idx = expert_indexes.astype(jnp.int32)
    idx0 = idx[..., 0].reshape(-1)
    idx1 = idx[..., 1].reshape(-1)
    counts = _sc_expert_counts(idx0, idx1)
    out = _tc_loss(counts, router_logits)
    return out[0, 0]
